# Initial kernel scaffold; baseline (speedup 1.0000x reference)
import jax
import jax.numpy as jnp
from jax.experimental import pallas as pl


def kernel(x, edge_index, edge_weight, W_ih0, W_hh0, b_ih0, b_hh0, ln0_g, ln0_b, W_ih1, W_hh1, b_ih1, b_hh1, ln1_g, ln1_b, gcn_W0, gcn_b0, gcn_W1, gcn_b1, fc_W, fc_b):
    raise NotImplementedError("write your pallas kernel here")


# speedup vs baseline: 3.3967x; 3.3967x over previous
import functools

import jax
import jax.numpy as jnp
from jax import lax
from jax.experimental import pallas as pl
from jax.experimental.pallas import tpu as pltpu
from jax.experimental.pallas import tpu_sc as plsc

_NB = 1280
_CHUNK = 128
_NTILES = 32


def _lstm_body(x_ref, wih0_ref, whh0t_ref, b0_ref, lng_ref, lnb_ref,
               wih1t_ref, whh1t_ref, b1_ref, w1_ref, h_ref, xw_ref):
    xb = x_ref[...]
    nb, t_len = xb.shape
    h = whh0t_ref.shape[0]
    lane_iota = lax.broadcasted_iota(jnp.int32, (1, t_len), 1)
    wih0 = wih0_ref[...]
    b0 = b0_ref[...]
    b1 = b1_ref[...]
    lng = lng_ref[...]
    lnb = lnb_ref[...]
    whh0t = whh0t_ref[...]
    wih1t = wih1t_ref[...]
    whh1t = whh1t_ref[...]

    def step(t, carry):
        h0, c0, h1, c1 = carry
        sel = (lane_iota == t).astype(xb.dtype)
        xt = jnp.sum(xb * sel, axis=1, keepdims=True)
        g0 = xt * wih0 + b0 + jnp.dot(h0, whh0t,
                                      preferred_element_type=jnp.float32)
        i0 = jax.nn.sigmoid(g0[:, :h])
        f0 = jax.nn.sigmoid(g0[:, h:2 * h])
        u0 = jnp.tanh(g0[:, 2 * h:3 * h])
        o0 = jax.nn.sigmoid(g0[:, 3 * h:])
        c0n = f0 * c0 + i0 * u0
        h0n = o0 * jnp.tanh(c0n)
        mu = jnp.mean(h0n, axis=1, keepdims=True)
        var = jnp.mean((h0n - mu) ** 2, axis=1, keepdims=True)
        xn = (h0n - mu) * lax.rsqrt(var + 1e-5) * lng + lnb
        g1 = (jnp.dot(xn, wih1t, preferred_element_type=jnp.float32)
              + jnp.dot(h1, whh1t, preferred_element_type=jnp.float32) + b1)
        i1 = jax.nn.sigmoid(g1[:, :h])
        f1 = jax.nn.sigmoid(g1[:, h:2 * h])
        u1 = jnp.tanh(g1[:, 2 * h:3 * h])
        o1 = jax.nn.sigmoid(g1[:, 3 * h:])
        c1n = f1 * c1 + i1 * u1
        h1n = o1 * jnp.tanh(c1n)
        return h0n, c0n, h1n, c1n

    z = jnp.zeros((nb, h), jnp.float32)
    _, _, h1, _ = lax.fori_loop(0, t_len, step, (z, z, z, z))
    h_ref[...] = h1
    xw_ref[...] = jnp.dot(h1, w1_ref[...], preferred_element_type=jnp.float32)


def _lstm_call(x_p, wih0, whh0t, b0, lng, lnb, wih1t, whh1t, b1, w1):
    n_pad, t_len = x_p.shape
    h = whh0t.shape[0]
    grid = (n_pad // _NB,)
    rep = lambda shape: pl.BlockSpec(shape, lambda i: (0,) * len(shape))
    return pl.pallas_call(
        _lstm_body,
        grid=grid,
        in_specs=[
            pl.BlockSpec((_NB, t_len), lambda i: (i, 0)),
            rep(wih0.shape), rep(whh0t.shape), rep(b0.shape),
            rep(lng.shape), rep(lnb.shape), rep(wih1t.shape),
            rep(whh1t.shape), rep(b1.shape), rep(w1.shape),
        ],
        out_specs=[pl.BlockSpec((_NB, h), lambda i: (i, 0)),
                   pl.BlockSpec((_NB, h), lambda i: (i, 0))],
        out_shape=[jax.ShapeDtypeStruct((n_pad, h), jnp.float32),
                   jax.ShapeDtypeStruct((n_pad, h), jnp.float32)],
    )(x_p, wih0, whh0t, b0, lng, lnb, wih1t, whh1t, b1, w1)


def _deg_body(n_pad, ept, col_hbm, w_hbm, z_hbm, out_hbm, col_v, w_v, deg_sh):
    c = lax.axis_index("c")
    s = lax.axis_index("s")
    wid = c * 16 + s
    sl = n_pad // 16
    pltpu.sync_copy(z_hbm, deg_sh.at[pl.ds(s * sl, sl)])
    plsc.subcore_barrier()

    def body(k, carry):
        base = wid * ept + k * _CHUNK
        pltpu.sync_copy(col_hbm.at[pl.ds(base, _CHUNK)], col_v)
        pltpu.sync_copy(w_hbm.at[pl.ds(base, _CHUNK)], w_v)
        pltpu.sync_copy(w_v, deg_sh.at[col_v], add=True)
        return carry

    lax.fori_loop(0, ept // _CHUNK, body, 0)
    plsc.subcore_barrier()
    pltpu.sync_copy(deg_sh.at[pl.ds(s * sl, sl)],
                    out_hbm.at[c, pl.ds(s * sl, sl)])


def _deg_call(col_p, w_p, n_pad):
    e_pad = col_p.shape[0]
    ept = e_pad // _NTILES
    sl = n_pad // 16
    z = jnp.zeros((sl,), jnp.float32)
    mesh = plsc.VectorSubcoreMesh(core_axis_name="c", subcore_axis_name="s")
    f = pl.kernel(
        functools.partial(_deg_body, n_pad, ept),
        out_type=jax.ShapeDtypeStruct((2, n_pad), jnp.float32),
        mesh=mesh,
        scratch_types=[
            pltpu.VMEM((_CHUNK,), jnp.int32),
            pltpu.VMEM((_CHUNK,), jnp.float32),
            pltpu.VMEM_SHARED((n_pad,), jnp.float32),
        ],
    )
    return f(col_p, w_p, z)


def _disy_body(degp_ref, xw_ref, dis_ref, y_ref):
    deg = degp_ref[0] + degp_ref[1] + 1.0
    dis = jnp.where(deg > 0, lax.rsqrt(jnp.where(deg > 0, deg, 1.0)), 0.0)
    dis_ref[...] = dis
    y_ref[...] = xw_ref[...] * dis


def _disy_call(degp_col, xw1):
    n_pad, h = xw1.shape
    return pl.pallas_call(
        _disy_body,
        out_shape=[jax.ShapeDtypeStruct((n_pad, 1), jnp.float32),
                   jax.ShapeDtypeStruct((n_pad, h), jnp.float32)],
    )(degp_col, xw1)


def _scat_body(n_pad, ept, h, row_hbm, col_hbm, w_hbm, y_hbm, z_hbm, out_hbm,
               row_v, col_v, w_v, rows_v, acc_sh):
    c = lax.axis_index("c")
    s = lax.axis_index("s")
    wid = c * 16 + s
    sl = n_pad // 16
    pltpu.sync_copy(z_hbm, acc_sh.at[pl.ds(s * sl, sl)])
    plsc.subcore_barrier()

    def chunk(k, carry):
        base = wid * ept + k * _CHUNK
        pltpu.sync_copy(row_hbm.at[pl.ds(base, _CHUNK)], row_v)
        pltpu.sync_copy(col_hbm.at[pl.ds(base, _CHUNK)], col_v)
        pltpu.sync_copy(w_hbm.at[pl.ds(base, _CHUNK)], w_v)
        pltpu.sync_copy(y_hbm.at[row_v], rows_v)

        def mul(g, cc):
            w16 = w_v[pl.ds(g * 16, 16)]
            for l in range(16):
                ws = w16[l]
                e = g * 16 + l
                for j in range(0, h, 16):
                    rows_v[e, pl.ds(j, 16)] = rows_v[e, pl.ds(j, 16)] * ws
            return cc

        lax.fori_loop(0, _CHUNK // 16, mul, 0)
        pltpu.sync_copy(rows_v, acc_sh.at[col_v], add=True)
        return carry

    lax.fori_loop(0, ept // _CHUNK, chunk, 0)
    plsc.subcore_barrier()
    pltpu.sync_copy(acc_sh.at[pl.ds(s * sl, sl)],
                    out_hbm.at[c, pl.ds(s * sl, sl)])


def _scat_call(row_p, col_p, w_p, y, n_pad):
    e_pad = row_p.shape[0]
    h = y.shape[1]
    ept = e_pad // _NTILES
    sl = n_pad // 16
    z = jnp.zeros((sl, h), jnp.float32)
    mesh = plsc.VectorSubcoreMesh(core_axis_name="c", subcore_axis_name="s")
    f = pl.kernel(
        functools.partial(_scat_body, n_pad, ept, h),
        out_type=jax.ShapeDtypeStruct((2, n_pad, h), jnp.float32),
        mesh=mesh,
        compiler_params=pltpu.CompilerParams(use_tc_tiling_on_sc=False),
        scratch_types=[
            pltpu.VMEM((_CHUNK,), jnp.int32),
            pltpu.VMEM((_CHUNK,), jnp.int32),
            pltpu.VMEM((_CHUNK,), jnp.float32),
            pltpu.VMEM((_CHUNK, h), jnp.float32),
            pltpu.VMEM_SHARED((n_pad, h), jnp.float32),
        ],
    )
    return f(row_p, col_p, w_p, y, z)


def _fin_body(accp_ref, y_ref, dis_ref, gin_ref, b1_ref, fwh_ref, fwm_ref,
              fb_ref, out_ref):
    conv = (accp_ref[0] + accp_ref[1] + y_ref[...]) * dis_ref[...] + b1_ref[...]
    g1 = jnp.where(conv > 0, conv, jnp.exp(jnp.minimum(conv, 0.0)) - 1.0)
    m = jnp.mean(g1, axis=1, keepdims=True)
    logits = (jnp.dot(gin_ref[...], fwh_ref[...],
                      preferred_element_type=jnp.float32)
              + m * fwm_ref[...] + fb_ref[...])
    lmax = jnp.max(logits, axis=1, keepdims=True)
    ex = jnp.exp(logits - lmax)
    lse = jnp.log(jnp.sum(ex, axis=1, keepdims=True))
    out_ref[...] = logits - lmax - lse


def _fin_call(accp, y, dis, gin, b1, fwh, fwm, fb):
    n_pad = y.shape[0]
    return pl.pallas_call(
        _fin_body,
        out_shape=jax.ShapeDtypeStruct((n_pad, 2), jnp.float32),
    )(accp, y, dis, gin, b1, fwh, fwm, fb)


def kernel(x, edge_index, edge_weight, W_ih0, W_hh0, b_ih0, b_hh0, ln0_g,
           ln0_b, W_ih1, W_hh1, b_ih1, b_hh1, ln1_g, ln1_b, gcn_W0, gcn_b0,
           gcn_W1, gcn_b1, fc_W, fc_b):
    n, t_len = x.shape
    h = W_hh0.shape[1]
    e = edge_weight.shape[0]

    n_pad = -(-n // _NB) * _NB
    x_p = jnp.pad(x, ((0, n_pad - n), (0, 0)))

    wih0 = W_ih0[:, 0][None, :]
    b0 = (b_ih0 + b_hh0)[None, :]
    b1 = (b_ih1 + b_hh1)[None, :]
    gin, xw1 = _lstm_call(x_p, wih0, W_hh0.T, b0, ln0_g[None, :],
                          ln0_b[None, :], W_ih1.T, W_hh1.T, b1, gcn_W1)

    grp = _NTILES * _CHUNK
    e_pad = -(-e // grp) * grp
    pad = e_pad - e
    pad_idx = (jnp.arange(pad, dtype=jnp.int32) * 97) % n
    row_p = jnp.concatenate([edge_index[0], pad_idx])
    col_p = jnp.concatenate([edge_index[1], pad_idx])
    w_p = jnp.concatenate([edge_weight, jnp.zeros((pad,), jnp.float32)])

    degp = _deg_call(col_p, w_p, n_pad)
    dis, y = _disy_call(degp[:, :, None], xw1)
    accp = _scat_call(row_p, col_p, w_p, y, n_pad)
    out = _fin_call(accp, y, dis, gin, gcn_b1[None, :],
                    fc_W[:h, :], fc_W[h:, :].reshape(1, 2), fc_b[None, :])
    return out[:n]


# speedup vs baseline: 10.1301x; 2.9823x over previous
import functools

import jax
import jax.numpy as jnp
from jax import lax
from jax.experimental import pallas as pl
from jax.experimental.pallas import tpu as pltpu
from jax.experimental.pallas import tpu_sc as plsc

_NB = 1280
_CHUNK = 128
_NTILES = 32


def _sig(x):
    return 0.5 * jnp.tanh(0.5 * x) + 0.5


def _lstm_body(xt_ref, w0_ref, w1_ref, rv_ref, wout_ref, ho_ref, xw_ref):
    t_len, nb = xt_ref.shape
    h = w0_ref.shape[0] // 4
    w0 = w0_ref[...]
    w1 = w1_ref[...]
    rv = rv_ref[...]
    ones = jnp.ones((1, nb), jnp.float32)

    def step(t, carry):
        h0, c0, h1, c1 = carry
        xt = xt_ref[pl.ds(t, 1), :]
        in0 = jnp.concatenate([h0, xt, ones], axis=0)
        g0 = jnp.dot(w0, in0, preferred_element_type=jnp.float32)
        c0n = _sig(g0[h:2 * h]) * c0 + _sig(g0[:h]) * jnp.tanh(g0[2 * h:3 * h])
        h0n = _sig(g0[3 * h:]) * jnp.tanh(c0n)
        mu = jnp.dot(rv, h0n, preferred_element_type=jnp.float32)
        xc = h0n - mu
        var = jnp.dot(rv, xc * xc, preferred_element_type=jnp.float32)
        xn = xc * lax.rsqrt(var + 1e-5)
        in1 = jnp.concatenate([xn, h1, ones], axis=0)
        g1 = jnp.dot(w1, in1, preferred_element_type=jnp.float32)
        c1n = _sig(g1[h:2 * h]) * c1 + _sig(g1[:h]) * jnp.tanh(g1[2 * h:3 * h])
        h1n = _sig(g1[3 * h:]) * jnp.tanh(c1n)
        return h0n, c0n, h1n, c1n

    z = jnp.zeros((h, nb), jnp.float32)
    _, _, h1, _ = lax.fori_loop(0, t_len, step, (z, z, z, z))
    ho_ref[...] = h1
    xw_ref[...] = jnp.dot(wout_ref[...], h1, preferred_element_type=jnp.float32)


def _lstm_call(xt_p, w0, w1, rv, wout):
    t_len, n_pad = xt_p.shape
    h = w0.shape[0] // 4
    grid = (n_pad // _NB,)
    rep = lambda shape: pl.BlockSpec(shape, lambda i: (0,) * len(shape))
    return pl.pallas_call(
        _lstm_body,
        grid=grid,
        in_specs=[
            pl.BlockSpec((t_len, _NB), lambda i: (0, i)),
            rep(w0.shape), rep(w1.shape), rep(rv.shape), rep(wout.shape),
        ],
        out_specs=[pl.BlockSpec((h, _NB), lambda i: (0, i)),
                   pl.BlockSpec((h, _NB), lambda i: (0, i))],
        out_shape=[jax.ShapeDtypeStruct((h, n_pad), jnp.float32),
                   jax.ShapeDtypeStruct((h, n_pad), jnp.float32)],
    )(xt_p, w0, w1, rv, wout)


def _deg_body(n_pad, ept, col_hbm, w_hbm, z_hbm, out_hbm, col_v, w_v, deg_sh):
    c = lax.axis_index("c")
    s = lax.axis_index("s")
    wid = c * 16 + s
    sl = n_pad // 16
    pltpu.sync_copy(z_hbm, deg_sh.at[pl.ds(s * sl, sl)])
    plsc.subcore_barrier()

    def body(k, carry):
        base = wid * ept + k * _CHUNK
        pltpu.sync_copy(col_hbm.at[pl.ds(base, _CHUNK)], col_v)
        pltpu.sync_copy(w_hbm.at[pl.ds(base, _CHUNK)], w_v)
        pltpu.sync_copy(w_v, deg_sh.at[col_v], add=True)
        return carry

    lax.fori_loop(0, ept // _CHUNK, body, 0)
    plsc.subcore_barrier()
    pltpu.sync_copy(deg_sh.at[pl.ds(s * sl, sl)],
                    out_hbm.at[c, pl.ds(s * sl, sl)])


def _deg_call(col_p, w_p, n_pad):
    e_pad = col_p.shape[0]
    ept = e_pad // _NTILES
    sl = n_pad // 16
    z = jnp.zeros((sl,), jnp.float32)
    mesh = plsc.VectorSubcoreMesh(core_axis_name="c", subcore_axis_name="s")
    f = pl.kernel(
        functools.partial(_deg_body, n_pad, ept),
        out_type=jax.ShapeDtypeStruct((2, n_pad), jnp.float32),
        mesh=mesh,
        scratch_types=[
            pltpu.VMEM((_CHUNK,), jnp.int32),
            pltpu.VMEM((_CHUNK,), jnp.float32),
            pltpu.VMEM_SHARED((n_pad,), jnp.float32),
        ],
    )
    return f(col_p, w_p, z)


def _disy_body(degp_ref, xw_ref, dis_ref, y_ref):
    deg = degp_ref[0] + degp_ref[1] + 1.0
    dis = jnp.where(deg > 0, lax.rsqrt(jnp.where(deg > 0, deg, 1.0)), 0.0)
    dis_ref[...] = dis
    y_ref[...] = xw_ref[...] * dis


def _disy_call(degp_col, xw1):
    n_pad, h = xw1.shape
    return pl.pallas_call(
        _disy_body,
        out_shape=[jax.ShapeDtypeStruct((n_pad, 1), jnp.float32),
                   jax.ShapeDtypeStruct((n_pad, h), jnp.float32)],
    )(degp_col, xw1)


def _scat_body(n_pad, ept, h, row_hbm, col_hbm, w_hbm, y_hbm, z_hbm, out_hbm,
               row_v, col_v, w_v, rows_v, acc_sh):
    c = lax.axis_index("c")
    s = lax.axis_index("s")
    wid = c * 16 + s
    sl = n_pad // 16
    pltpu.sync_copy(z_hbm, acc_sh.at[pl.ds(s * sl, sl)])
    plsc.subcore_barrier()

    def chunk(k, carry):
        base = wid * ept + k * _CHUNK
        pltpu.sync_copy(row_hbm.at[pl.ds(base, _CHUNK)], row_v)
        pltpu.sync_copy(col_hbm.at[pl.ds(base, _CHUNK)], col_v)
        pltpu.sync_copy(w_hbm.at[pl.ds(base, _CHUNK)], w_v)
        pltpu.sync_copy(y_hbm.at[row_v], rows_v)

        def mul(g, cc):
            w16 = w_v[pl.ds(g * 16, 16)]
            for l in range(16):
                ws = w16[l]
                e = g * 16 + l
                for j in range(0, h, 16):
                    rows_v[e, pl.ds(j, 16)] = rows_v[e, pl.ds(j, 16)] * ws
            return cc

        lax.fori_loop(0, _CHUNK // 16, mul, 0)
        pltpu.sync_copy(rows_v, acc_sh.at[col_v], add=True)
        return carry

    lax.fori_loop(0, ept // _CHUNK, chunk, 0)
    plsc.subcore_barrier()
    pltpu.sync_copy(acc_sh.at[pl.ds(s * sl, sl)],
                    out_hbm.at[c, pl.ds(s * sl, sl)])


def _scat_call(row_p, col_p, w_p, y, n_pad):
    e_pad = row_p.shape[0]
    h = y.shape[1]
    ept = e_pad // _NTILES
    sl = n_pad // 16
    z = jnp.zeros((sl, h), jnp.float32)
    mesh = plsc.VectorSubcoreMesh(core_axis_name="c", subcore_axis_name="s")
    f = pl.kernel(
        functools.partial(_scat_body, n_pad, ept, h),
        out_type=jax.ShapeDtypeStruct((2, n_pad, h), jnp.float32),
        mesh=mesh,
        compiler_params=pltpu.CompilerParams(use_tc_tiling_on_sc=False),
        scratch_types=[
            pltpu.VMEM((_CHUNK,), jnp.int32),
            pltpu.VMEM((_CHUNK,), jnp.int32),
            pltpu.VMEM((_CHUNK,), jnp.float32),
            pltpu.VMEM((_CHUNK, h), jnp.float32),
            pltpu.VMEM_SHARED((n_pad, h), jnp.float32),
        ],
    )
    return f(row_p, col_p, w_p, y, z)


def _fin_body(accp_ref, y_ref, dis_ref, gin_ref, b1_ref, fwh_ref, fwm_ref,
              fb_ref, out_ref):
    conv = (accp_ref[0] + accp_ref[1] + y_ref[...]) * dis_ref[...] + b1_ref[...]
    g1 = jnp.where(conv > 0, conv, jnp.exp(jnp.minimum(conv, 0.0)) - 1.0)
    m = jnp.mean(g1, axis=1, keepdims=True)
    logits = (jnp.dot(gin_ref[...], fwh_ref[...],
                      preferred_element_type=jnp.float32)
              + m * fwm_ref[...] + fb_ref[...])
    lmax = jnp.max(logits, axis=1, keepdims=True)
    ex = jnp.exp(logits - lmax)
    lse = jnp.log(jnp.sum(ex, axis=1, keepdims=True))
    out_ref[...] = logits - lmax - lse


def _fin_call(accp, y, dis, gin, b1, fwh, fwm, fb):
    n_pad = y.shape[0]
    return pl.pallas_call(
        _fin_body,
        out_shape=jax.ShapeDtypeStruct((n_pad, 2), jnp.float32),
    )(accp, y, dis, gin, b1, fwh, fwm, fb)


def kernel(x, edge_index, edge_weight, W_ih0, W_hh0, b_ih0, b_hh0, ln0_g,
           ln0_b, W_ih1, W_hh1, b_ih1, b_hh1, ln1_g, ln1_b, gcn_W0, gcn_b0,
           gcn_W1, gcn_b1, fc_W, fc_b):
    n, t_len = x.shape
    h = W_hh0.shape[1]
    e = edge_weight.shape[0]

    n_pad = -(-n // _NB) * _NB
    xt_p = jnp.pad(x, ((0, n_pad - n), (0, 0))).T

    w0 = jnp.concatenate(
        [W_hh0, W_ih0[:, :1], (b_ih0 + b_hh0)[:, None]], axis=1)
    b1p = b_ih1 + b_hh1 + W_ih1 @ ln0_b
    w1 = jnp.concatenate(
        [W_ih1 * ln0_g[None, :], W_hh1, b1p[:, None]], axis=1)
    rv = jnp.full((1, h), 1.0 / h, jnp.float32)
    gin_t, xw1_t = _lstm_call(xt_p, w0, w1, rv, gcn_W1.T)
    gin = gin_t.T
    xw1 = xw1_t.T

    grp = _NTILES * _CHUNK
    e_pad = -(-e // grp) * grp
    pad = e_pad - e
    pad_idx = (jnp.arange(pad, dtype=jnp.int32) * 97) % n
    row_p = jnp.concatenate([edge_index[0], pad_idx])
    col_p = jnp.concatenate([edge_index[1], pad_idx])
    w_p = jnp.concatenate([edge_weight, jnp.zeros((pad,), jnp.float32)])

    degp = _deg_call(col_p, w_p, n_pad)
    dis, y = _disy_call(degp[:, :, None], xw1)
    accp = _scat_call(row_p, col_p, w_p, y, n_pad)
    out = _fin_call(accp, y, dis, gin, gcn_b1[None, :],
                    fc_W[:h, :], fc_W[h:, :].reshape(1, 2), fc_b[None, :])
    return out[:n]


# speedup vs baseline: 10.5175x; 1.0382x over previous
import functools

import jax
import jax.numpy as jnp
from jax import lax
from jax.experimental import pallas as pl
from jax.experimental.pallas import tpu as pltpu
from jax.experimental.pallas import tpu_sc as plsc

_NB = 1280
_CHUNK = 128
_NTILES = 32


def _sig(x):
    return 0.5 * jnp.tanh(0.5 * x) + 0.5


def _lstm_body(xt_ref, w0_ref, w1_ref, rv_ref, wout_ref, ho_ref, xw_ref):
    t_len, nb = xt_ref.shape
    h = w0_ref.shape[0] // 4
    w0 = w0_ref[...]
    w1 = w1_ref[...]
    rv = rv_ref[...]
    ones = jnp.ones((1, nb), jnp.float32)

    def layer0(t, h0, c0):
        xt = xt_ref[pl.ds(t, 1), :]
        in0 = jnp.concatenate([h0, xt, ones], axis=0)
        g0 = jnp.dot(w0, in0, preferred_element_type=jnp.float32)
        c0n = _sig(g0[h:2 * h]) * c0 + _sig(g0[:h]) * jnp.tanh(g0[2 * h:3 * h])
        h0n = _sig(g0[3 * h:]) * jnp.tanh(c0n)
        mu = jnp.dot(rv, h0n, preferred_element_type=jnp.float32)
        xc = h0n - mu
        var = jnp.dot(rv, xc * xc, preferred_element_type=jnp.float32)
        xn = xc * lax.rsqrt(var + 1e-5)
        return h0n, c0n, xn

    def layer1(xn, h1, c1):
        in1 = jnp.concatenate([xn, h1, ones], axis=0)
        g1 = jnp.dot(w1, in1, preferred_element_type=jnp.float32)
        c1n = _sig(g1[h:2 * h]) * c1 + _sig(g1[:h]) * jnp.tanh(g1[2 * h:3 * h])
        h1n = _sig(g1[3 * h:]) * jnp.tanh(c1n)
        return h1n, c1n

    def step(t, carry):
        h0, c0, h1, c1, xn = carry
        h0n, c0n, xn_new = layer0(t, h0, c0)
        h1n, c1n = layer1(xn, h1, c1)
        return h0n, c0n, h1n, c1n, xn_new

    z = jnp.zeros((h, nb), jnp.float32)
    h0, c0, xn = layer0(0, z, z)
    _, _, h1, c1, xn = lax.fori_loop(1, t_len, step, (h0, c0, z, z, xn))
    h1, _ = layer1(xn, h1, c1)
    ho_ref[...] = h1
    xw_ref[...] = jnp.dot(wout_ref[...], h1, preferred_element_type=jnp.float32)


def _lstm_call(xt_p, w0, w1, rv, wout):
    t_len, n_pad = xt_p.shape
    h = w0.shape[0] // 4
    grid = (n_pad // _NB,)
    rep = lambda shape: pl.BlockSpec(shape, lambda i: (0,) * len(shape))
    return pl.pallas_call(
        _lstm_body,
        grid=grid,
        in_specs=[
            pl.BlockSpec((t_len, _NB), lambda i: (0, i)),
            rep(w0.shape), rep(w1.shape), rep(rv.shape), rep(wout.shape),
        ],
        out_specs=[pl.BlockSpec((h, _NB), lambda i: (0, i)),
                   pl.BlockSpec((h, _NB), lambda i: (0, i))],
        out_shape=[jax.ShapeDtypeStruct((h, n_pad), jnp.float32),
                   jax.ShapeDtypeStruct((h, n_pad), jnp.float32)],
    )(xt_p, w0, w1, rv, wout)


def _deg_body(n_pad, ept, col_hbm, w_hbm, z_hbm, out_hbm, col_v, w_v, deg_sh):
    c = lax.axis_index("c")
    s = lax.axis_index("s")
    wid = c * 16 + s
    sl = n_pad // 16
    pltpu.sync_copy(z_hbm, deg_sh.at[pl.ds(s * sl, sl)])
    plsc.subcore_barrier()

    def body(k, carry):
        base = wid * ept + k * _CHUNK
        pltpu.sync_copy(col_hbm.at[pl.ds(base, _CHUNK)], col_v)
        pltpu.sync_copy(w_hbm.at[pl.ds(base, _CHUNK)], w_v)
        pltpu.sync_copy(w_v, deg_sh.at[col_v], add=True)
        return carry

    lax.fori_loop(0, ept // _CHUNK, body, 0)
    plsc.subcore_barrier()
    pltpu.sync_copy(deg_sh.at[pl.ds(s * sl, sl)],
                    out_hbm.at[c, pl.ds(s * sl, sl)])


def _deg_call(col_p, w_p, n_pad):
    e_pad = col_p.shape[0]
    ept = e_pad // _NTILES
    sl = n_pad // 16
    z = jnp.zeros((sl,), jnp.float32)
    mesh = plsc.VectorSubcoreMesh(core_axis_name="c", subcore_axis_name="s")
    f = pl.kernel(
        functools.partial(_deg_body, n_pad, ept),
        out_type=jax.ShapeDtypeStruct((2, n_pad), jnp.float32),
        mesh=mesh,
        scratch_types=[
            pltpu.VMEM((_CHUNK,), jnp.int32),
            pltpu.VMEM((_CHUNK,), jnp.float32),
            pltpu.VMEM_SHARED((n_pad,), jnp.float32),
        ],
    )
    return f(col_p, w_p, z)


def _disy_body(degp_ref, xw_ref, dis_ref, y_ref):
    deg = degp_ref[0] + degp_ref[1] + 1.0
    dis = jnp.where(deg > 0, lax.rsqrt(jnp.where(deg > 0, deg, 1.0)), 0.0)
    dis_ref[...] = dis
    y_ref[...] = xw_ref[...] * dis


def _disy_call(degp_col, xw1):
    n_pad, h = xw1.shape
    return pl.pallas_call(
        _disy_body,
        out_shape=[jax.ShapeDtypeStruct((n_pad, 1), jnp.float32),
                   jax.ShapeDtypeStruct((n_pad, h), jnp.float32)],
    )(degp_col, xw1)


def _scat_body(n_pad, ept, h, row_hbm, col_hbm, w_hbm, y_hbm, z_hbm, out_hbm,
               row_v, col_v, w_v, rows_v, acc_sh):
    c = lax.axis_index("c")
    s = lax.axis_index("s")
    wid = c * 16 + s
    sl = n_pad // 16
    pltpu.sync_copy(z_hbm, acc_sh.at[pl.ds(s * sl, sl)])
    plsc.subcore_barrier()

    def chunk(k, carry):
        base = wid * ept + k * _CHUNK
        pltpu.sync_copy(row_hbm.at[pl.ds(base, _CHUNK)], row_v)
        pltpu.sync_copy(col_hbm.at[pl.ds(base, _CHUNK)], col_v)
        pltpu.sync_copy(w_hbm.at[pl.ds(base, _CHUNK)], w_v)
        pltpu.sync_copy(y_hbm.at[row_v], rows_v)

        def mul(g, cc):
            w16 = w_v[pl.ds(g * 16, 16)]
            for l in range(16):
                ws = w16[l]
                e = g * 16 + l
                for j in range(0, h, 16):
                    rows_v[e, pl.ds(j, 16)] = rows_v[e, pl.ds(j, 16)] * ws
            return cc

        lax.fori_loop(0, _CHUNK // 16, mul, 0)
        pltpu.sync_copy(rows_v, acc_sh.at[col_v], add=True)
        return carry

    lax.fori_loop(0, ept // _CHUNK, chunk, 0)
    plsc.subcore_barrier()
    pltpu.sync_copy(acc_sh.at[pl.ds(s * sl, sl)],
                    out_hbm.at[c, pl.ds(s * sl, sl)])


def _scat_call(row_p, col_p, w_p, y, n_pad):
    e_pad = row_p.shape[0]
    h = y.shape[1]
    ept = e_pad // _NTILES
    sl = n_pad // 16
    z = jnp.zeros((sl, h), jnp.float32)
    mesh = plsc.VectorSubcoreMesh(core_axis_name="c", subcore_axis_name="s")
    f = pl.kernel(
        functools.partial(_scat_body, n_pad, ept, h),
        out_type=jax.ShapeDtypeStruct((2, n_pad, h), jnp.float32),
        mesh=mesh,
        compiler_params=pltpu.CompilerParams(use_tc_tiling_on_sc=False),
        scratch_types=[
            pltpu.VMEM((_CHUNK,), jnp.int32),
            pltpu.VMEM((_CHUNK,), jnp.int32),
            pltpu.VMEM((_CHUNK,), jnp.float32),
            pltpu.VMEM((_CHUNK, h), jnp.float32),
            pltpu.VMEM_SHARED((n_pad, h), jnp.float32),
        ],
    )
    return f(row_p, col_p, w_p, y, z)


def _fin_body(accp_ref, y_ref, dis_ref, gin_ref, b1_ref, fwh_ref, fwm_ref,
              fb_ref, out_ref):
    conv = (accp_ref[0] + accp_ref[1] + y_ref[...]) * dis_ref[...] + b1_ref[...]
    g1 = jnp.where(conv > 0, conv, jnp.exp(jnp.minimum(conv, 0.0)) - 1.0)
    m = jnp.mean(g1, axis=1, keepdims=True)
    logits = (jnp.dot(gin_ref[...], fwh_ref[...],
                      preferred_element_type=jnp.float32)
              + m * fwm_ref[...] + fb_ref[...])
    lmax = jnp.max(logits, axis=1, keepdims=True)
    ex = jnp.exp(logits - lmax)
    lse = jnp.log(jnp.sum(ex, axis=1, keepdims=True))
    out_ref[...] = logits - lmax - lse


def _fin_call(accp, y, dis, gin, b1, fwh, fwm, fb):
    n_pad = y.shape[0]
    return pl.pallas_call(
        _fin_body,
        out_shape=jax.ShapeDtypeStruct((n_pad, 2), jnp.float32),
    )(accp, y, dis, gin, b1, fwh, fwm, fb)


def kernel(x, edge_index, edge_weight, W_ih0, W_hh0, b_ih0, b_hh0, ln0_g,
           ln0_b, W_ih1, W_hh1, b_ih1, b_hh1, ln1_g, ln1_b, gcn_W0, gcn_b0,
           gcn_W1, gcn_b1, fc_W, fc_b):
    n, t_len = x.shape
    h = W_hh0.shape[1]
    e = edge_weight.shape[0]

    n_pad = -(-n // _NB) * _NB
    xt_p = jnp.pad(x, ((0, n_pad - n), (0, 0))).T

    w0 = jnp.concatenate(
        [W_hh0, W_ih0[:, :1], (b_ih0 + b_hh0)[:, None]], axis=1)
    b1p = b_ih1 + b_hh1 + W_ih1 @ ln0_b
    w1 = jnp.concatenate(
        [W_ih1 * ln0_g[None, :], W_hh1, b1p[:, None]], axis=1)
    rv = jnp.full((1, h), 1.0 / h, jnp.float32)
    gin_t, xw1_t = _lstm_call(xt_p, w0, w1, rv, gcn_W1.T)
    gin = gin_t.T
    xw1 = xw1_t.T

    grp = _NTILES * _CHUNK
    e_pad = -(-e // grp) * grp
    pad = e_pad - e
    pad_idx = (jnp.arange(pad, dtype=jnp.int32) * 97) % n
    row_p = jnp.concatenate([edge_index[0], pad_idx])
    col_p = jnp.concatenate([edge_index[1], pad_idx])
    w_p = jnp.concatenate([edge_weight, jnp.zeros((pad,), jnp.float32)])

    degp = _deg_call(col_p, w_p, n_pad)
    dis, y = _disy_call(degp[:, :, None], xw1)
    accp = _scat_call(row_p, col_p, w_p, y, n_pad)
    out = _fin_call(accp, y, dis, gin, gcn_b1[None, :],
                    fc_W[:h, :], fc_W[h:, :].reshape(1, 2), fc_b[None, :])
    return out[:n]


# speedup vs baseline: 12.3097x; 1.1704x over previous
import functools

import jax
import jax.numpy as jnp
from jax import lax
from jax.experimental import pallas as pl
from jax.experimental.pallas import tpu as pltpu
from jax.experimental.pallas import tpu_sc as plsc

_NB = 1280
_CHUNK = 128
_NTILES = 32


def _sig(x):
    return 0.5 * jnp.tanh(0.5 * x) + 0.5


def _lstm_body(xt_ref, w0_ref, w1_ref, rv_ref, wout_ref, ho_ref, xw_ref):
    t_len, nb = xt_ref.shape
    h = w0_ref.shape[0] // 4
    w0 = w0_ref[...]
    w1 = w1_ref[...]
    rv = rv_ref[...]
    ones = jnp.ones((1, nb), jnp.float32)

    def layer0(t, h0, c0):
        xt = xt_ref[pl.ds(t, 1), :]
        in0 = jnp.concatenate([h0, xt, ones], axis=0)
        g0 = jnp.dot(w0, in0, preferred_element_type=jnp.float32)
        c0n = _sig(g0[h:2 * h]) * c0 + _sig(g0[:h]) * jnp.tanh(g0[2 * h:3 * h])
        h0n = _sig(g0[3 * h:]) * jnp.tanh(c0n)
        mu = jnp.dot(rv, h0n, preferred_element_type=jnp.float32)
        xc = h0n - mu
        var = jnp.dot(rv, xc * xc, preferred_element_type=jnp.float32)
        xn = xc * lax.rsqrt(var + 1e-5)
        return h0n, c0n, xn

    def layer1(xn, h1, c1):
        in1 = jnp.concatenate([xn, h1, ones], axis=0)
        g1 = jnp.dot(w1, in1, preferred_element_type=jnp.float32)
        c1n = _sig(g1[h:2 * h]) * c1 + _sig(g1[:h]) * jnp.tanh(g1[2 * h:3 * h])
        h1n = _sig(g1[3 * h:]) * jnp.tanh(c1n)
        return h1n, c1n

    def step(t, carry):
        h0, c0, h1, c1, xn = carry
        h0n, c0n, xn_new = layer0(t, h0, c0)
        h1n, c1n = layer1(xn, h1, c1)
        return h0n, c0n, h1n, c1n, xn_new

    z = jnp.zeros((h, nb), jnp.float32)
    h0, c0, xn = layer0(0, z, z)
    _, _, h1, c1, xn = lax.fori_loop(1, t_len, step, (h0, c0, z, z, xn))
    h1, _ = layer1(xn, h1, c1)
    ho_ref[...] = h1
    xw_ref[...] = jnp.dot(wout_ref[...], h1, preferred_element_type=jnp.float32)


def _lstm_call(xt_p, w0, w1, rv, wout):
    t_len, n_pad = xt_p.shape
    h = w0.shape[0] // 4
    grid = (n_pad // _NB,)
    rep = lambda shape: pl.BlockSpec(shape, lambda i: (0,) * len(shape))
    return pl.pallas_call(
        _lstm_body,
        grid=grid,
        in_specs=[
            pl.BlockSpec((t_len, _NB), lambda i: (0, i)),
            rep(w0.shape), rep(w1.shape), rep(rv.shape), rep(wout.shape),
        ],
        out_specs=[pl.BlockSpec((h, _NB), lambda i: (0, i)),
                   pl.BlockSpec((h, _NB), lambda i: (0, i))],
        out_shape=[jax.ShapeDtypeStruct((h, n_pad), jnp.float32),
                   jax.ShapeDtypeStruct((h, n_pad), jnp.float32)],
    )(xt_p, w0, w1, rv, wout)


def _deg_body(n_pad, nch, edata_hbm, w_hbm, z_hbm, out_hbm,
              ed_all, w_all, deg_sh, sed, sw, sdma):
    c = lax.axis_index("c")
    s = lax.axis_index("s")
    wid = c * 16 + s
    sl = n_pad // 16
    pltpu.sync_copy(z_hbm, deg_sh.at[pl.ds(s * sl, sl)])
    pltpu.async_copy(edata_hbm.at[pl.ds(wid * nch, nch)], ed_all, sed)
    pltpu.async_copy(w_hbm.at[pl.ds(wid * nch, nch)], w_all, sw)
    pltpu.make_async_copy(edata_hbm.at[pl.ds(0, nch)], ed_all, sed).wait()
    pltpu.make_async_copy(w_hbm.at[pl.ds(0, nch)], w_all, sw).wait()
    plsc.subcore_barrier()

    def issue(k):
        pltpu.async_copy(w_all.at[k], deg_sh.at[ed_all.at[k, 1]], sdma,
                         add=True)

    def drain():
        pltpu.make_async_copy(w_all.at[0], deg_sh.at[ed_all.at[0, 1]],
                              sdma).wait()

    issue(0)
    issue(1)

    def body(k, carry):
        issue(k)
        drain()
        return carry

    lax.fori_loop(2, nch, body, 0)
    drain()
    drain()
    plsc.subcore_barrier()
    pltpu.sync_copy(deg_sh.at[pl.ds(s * sl, sl)],
                    out_hbm.at[c, pl.ds(s * sl, sl)])


def _deg_call(edata, wdata, n_pad):
    nch = edata.shape[0] // _NTILES
    sl = n_pad // 16
    z = jnp.zeros((sl,), jnp.float32)
    mesh = plsc.VectorSubcoreMesh(core_axis_name="c", subcore_axis_name="s")
    f = pl.kernel(
        functools.partial(_deg_body, n_pad, nch),
        out_type=jax.ShapeDtypeStruct((2, n_pad), jnp.float32),
        mesh=mesh,
        compiler_params=pltpu.CompilerParams(use_tc_tiling_on_sc=False),
        scratch_types=[
            pltpu.VMEM((nch, 2, _CHUNK), jnp.int32),
            pltpu.VMEM((nch, _CHUNK), jnp.float32),
            pltpu.VMEM_SHARED((n_pad,), jnp.float32),
            pltpu.SemaphoreType.DMA,
            pltpu.SemaphoreType.DMA,
            pltpu.SemaphoreType.DMA,
        ],
    )
    return f(edata, wdata, z)


def _disy_body(degp_ref, xw_ref, dis_ref, y_ref):
    deg = degp_ref[0] + degp_ref[1] + 1.0
    dis = jnp.where(deg > 0, lax.rsqrt(jnp.where(deg > 0, deg, 1.0)), 0.0)
    dis_ref[...] = dis
    y_ref[...] = xw_ref[...] * dis


def _disy_call(degp_col, xw1):
    n_pad, h = xw1.shape
    return pl.pallas_call(
        _disy_body,
        out_shape=[jax.ShapeDtypeStruct((n_pad, 1), jnp.float32),
                   jax.ShapeDtypeStruct((n_pad, h), jnp.float32)],
    )(degp_col, xw1)


def _scat_body(n_pad, nch, h, edata_hbm, w_hbm, y_hbm, z_hbm, out_hbm,
               ed_all, w_all, rb0, rb1, rb2, y_sh, acc_sh,
               sed, sw, sg0, sg1, sg2, ss0, ss1, ss2):
    c = lax.axis_index("c")
    s = lax.axis_index("s")
    wid = c * 16 + s
    sl = n_pad // 16
    rbs = (rb0, rb1, rb2)
    sgs = (sg0, sg1, sg2)
    sss = (ss0, ss1, ss2)
    pltpu.sync_copy(z_hbm, acc_sh.at[pl.ds(s * sl, sl)])
    pltpu.sync_copy(y_hbm.at[pl.ds(s * sl, sl)], y_sh.at[pl.ds(s * sl, sl)])
    pltpu.async_copy(edata_hbm.at[pl.ds(wid * nch, nch)], ed_all, sed)
    pltpu.async_copy(w_hbm.at[pl.ds(wid * nch, nch)], w_all, sw)
    pltpu.make_async_copy(edata_hbm.at[pl.ds(0, nch)], ed_all, sed).wait()
    pltpu.make_async_copy(w_hbm.at[pl.ds(0, nch)], w_all, sw).wait()
    plsc.subcore_barrier()

    def gather(k, b):
        return pltpu.async_copy(y_sh.at[ed_all.at[k, 0]], rbs[b], sgs[b])

    def gwait(b):
        pltpu.make_async_copy(y_sh.at[ed_all.at[0, 0]], rbs[b], sgs[b]).wait()

    def swait(b):
        pltpu.make_async_copy(rbs[b], acc_sh.at[ed_all.at[0, 1]],
                              sss[b]).wait()

    def mul(k, b):
        for g in range(_CHUNK // 16):
            wv = w_all[k, pl.ds(g * 16, 16)]
            for l in range(16):
                e = g * 16 + l
                ws = wv[l]
                for j in range(0, h, 16):
                    rbs[b][e, pl.ds(j, 16)] = rbs[b][e, pl.ds(j, 16)] * ws

    def scat(k, b):
        return pltpu.async_copy(rbs[b], acc_sh.at[ed_all.at[k, 1]], sss[b],
                                add=True)

    gather(0, 0)
    for b in range(3):
        if b == 2:
            swait(0)
        gather(b + 1, (b + 1) % 3)
        gwait(b)
        mul(b, b)
        scat(b, b)

    def mid(g, carry):
        for b in range(3):
            k = g * 3 + b
            bn = (b + 1) % 3
            swait(bn)
            gather(k + 1, bn)
            gwait(b)
            mul(k, b)
            scat(k, b)
        return carry

    lax.fori_loop(1, nch // 3 - 1, mid, 0)
    for b in range(3):
        k = nch - 3 + b
        bn = (b + 1) % 3
        if b < 2:
            swait(bn)
            gather(k + 1, bn)
        gwait(b)
        mul(k, b)
        scat(k, b)
    swait((nch - 3) % 3)
    swait((nch - 2) % 3)
    swait((nch - 1) % 3)
    plsc.subcore_barrier()
    pltpu.sync_copy(acc_sh.at[pl.ds(s * sl, sl)],
                    out_hbm.at[c, pl.ds(s * sl, sl)])


def _scat_call(edata, wdata, y, n_pad):
    nch = edata.shape[0] // _NTILES
    h = y.shape[1]
    sl = n_pad // 16
    z = jnp.zeros((sl, h), jnp.float32)
    mesh = plsc.VectorSubcoreMesh(core_axis_name="c", subcore_axis_name="s")
    f = pl.kernel(
        functools.partial(_scat_body, n_pad, nch, h),
        out_type=jax.ShapeDtypeStruct((2, n_pad, h), jnp.float32),
        mesh=mesh,
        compiler_params=pltpu.CompilerParams(use_tc_tiling_on_sc=False),
        scratch_types=[
            pltpu.VMEM((nch, 2, _CHUNK), jnp.int32),
            pltpu.VMEM((nch, _CHUNK), jnp.float32),
            pltpu.VMEM((_CHUNK, h), jnp.float32),
            pltpu.VMEM((_CHUNK, h), jnp.float32),
            pltpu.VMEM((_CHUNK, h), jnp.float32),
            pltpu.VMEM_SHARED((n_pad, h), jnp.float32),
            pltpu.VMEM_SHARED((n_pad, h), jnp.float32),
            pltpu.SemaphoreType.DMA,
            pltpu.SemaphoreType.DMA,
            pltpu.SemaphoreType.DMA,
            pltpu.SemaphoreType.DMA,
            pltpu.SemaphoreType.DMA,
            pltpu.SemaphoreType.DMA,
            pltpu.SemaphoreType.DMA,
            pltpu.SemaphoreType.DMA,
        ],
    )
    return f(edata, wdata, y, z)


def _fin_body(accp_ref, y_ref, dis_ref, gin_ref, b1_ref, fwh_ref, fwm_ref,
              fb_ref, out_ref):
    conv = (accp_ref[0] + accp_ref[1] + y_ref[...]) * dis_ref[...] + b1_ref[...]
    g1 = jnp.where(conv > 0, conv, jnp.exp(jnp.minimum(conv, 0.0)) - 1.0)
    m = jnp.mean(g1, axis=1, keepdims=True)
    logits = (jnp.dot(gin_ref[...], fwh_ref[...],
                      preferred_element_type=jnp.float32)
              + m * fwm_ref[...] + fb_ref[...])
    lmax = jnp.max(logits, axis=1, keepdims=True)
    ex = jnp.exp(logits - lmax)
    lse = jnp.log(jnp.sum(ex, axis=1, keepdims=True))
    out_ref[...] = logits - lmax - lse


def _fin_call(accp, y, dis, gin, b1, fwh, fwm, fb):
    n_pad = y.shape[0]
    return pl.pallas_call(
        _fin_body,
        out_shape=jax.ShapeDtypeStruct((n_pad, 2), jnp.float32),
    )(accp, y, dis, gin, b1, fwh, fwm, fb)


def kernel(x, edge_index, edge_weight, W_ih0, W_hh0, b_ih0, b_hh0, ln0_g,
           ln0_b, W_ih1, W_hh1, b_ih1, b_hh1, ln1_g, ln1_b, gcn_W0, gcn_b0,
           gcn_W1, gcn_b1, fc_W, fc_b):
    n, t_len = x.shape
    h = W_hh0.shape[1]
    e = edge_weight.shape[0]

    n_pad = -(-n // _NB) * _NB
    xt_p = jnp.pad(x, ((0, n_pad - n), (0, 0))).T

    w0 = jnp.concatenate(
        [W_hh0, W_ih0[:, :1], (b_ih0 + b_hh0)[:, None]], axis=1)
    b1p = b_ih1 + b_hh1 + W_ih1 @ ln0_b
    w1 = jnp.concatenate(
        [W_ih1 * ln0_g[None, :], W_hh1, b1p[:, None]], axis=1)
    rv = jnp.full((1, h), 1.0 / h, jnp.float32)
    gin_t, xw1_t = _lstm_call(xt_p, w0, w1, rv, gcn_W1.T)
    gin = gin_t.T
    xw1 = xw1_t.T

    grp = _NTILES * _CHUNK
    nch = -(-(-(-e // grp)) // 3) * 3
    e_pad = nch * grp
    pad = e_pad - e
    pad_idx = (jnp.arange(pad, dtype=jnp.int32) * 97) % n
    row_p = jnp.concatenate([edge_index[0], pad_idx])
    col_p = jnp.concatenate([edge_index[1], pad_idx])
    w_p = jnp.concatenate([edge_weight, jnp.zeros((pad,), jnp.float32)])
    edata = jnp.stack([row_p, col_p])
    edata = edata.reshape(2, _NTILES * nch, _CHUNK).transpose(1, 0, 2)
    wdata = w_p.reshape(_NTILES * nch, _CHUNK)

    degp = _deg_call(edata, wdata, n_pad)
    dis, y = _disy_call(degp[:, :, None], xw1)
    accp = _scat_call(edata, wdata, y, n_pad)
    out = _fin_call(accp, y, dis, gin, gcn_b1[None, :],
                    fc_W[:h, :], fc_W[h:, :].reshape(1, 2), fc_b[None, :])
    return out[:n]
